# baseline (speedup 1.0000x reference)
import functools

import jax
import jax.numpy as jnp
from jax import lax
from jax.experimental import pallas as pl
from jax.experimental.pallas import tpu as pltpu
from jax.experimental.pallas import tpu_sc as plsc

L = 16
L2 = 32
NW = 32
G = 128
C = 256


@functools.cache
def _make_sc_kernel(Ep, N, D):
    per_w = Ep // NW
    n_chunks = per_w // C
    JW = D // L2
    assert per_w % C == 0 and D % L2 == 0

    mesh = plsc.VectorSubcoreMesh(core_axis_name="c", subcore_axis_name="s")

    @functools.partial(
        pl.kernel,
        mesh=mesh,
        out_type=jax.ShapeDtypeStruct((NW, per_w), jnp.float32),
        compiler_params=pltpu.CompilerParams(needs_layout_passes=False,
                                             use_tc_tiling_on_sc=False),
        scratch_types=[
            pltpu.VMEM((2, per_w // G, G), jnp.int32),
            pltpu.VMEM((2, C, D), jnp.bfloat16),
            pltpu.VMEM((2, C, D), jnp.bfloat16),
            pltpu.VMEM((per_w,), jnp.float32),
            pltpu.SemaphoreType.DMA((2,)),
            pltpu.SemaphoreType.DMA((2,)),
        ],
    )
    def sc_kernel(hsrc_hbm, hdst_hbm, eidx_hbm, out_hbm,
                  eidx_v, srow_v, drow_v, score_v, ssem, dsem):
        wid = lax.axis_index("s") * 2 + lax.axis_index("c")
        pltpu.sync_copy(eidx_hbm.at[:, wid], eidx_v)

        last_lane = lax.iota(jnp.int32, L) == L - 1

        def start(ci, b):
            for h in range(2):
                pltpu.async_copy(hsrc_hbm.at[eidx_v.at[0, 2 * ci + h]],
                                 srow_v.at[b, pl.ds(h * G, G)], ssem.at[b])
                pltpu.async_copy(hdst_hbm.at[eidx_v.at[1, 2 * ci + h]],
                                 drow_v.at[b, pl.ds(h * G, G)], dsem.at[b])

        def wait(ci, b):
            for h in range(2):
                pltpu.make_async_copy(
                    hsrc_hbm.at[eidx_v.at[0, 2 * ci + h]],
                    srow_v.at[b, pl.ds(h * G, G)], ssem.at[b]).wait()
                pltpu.make_async_copy(
                    hdst_hbm.at[eidx_v.at[1, 2 * ci + h]],
                    drow_v.at[b, pl.ds(h * G, G)], dsem.at[b]).wait()

        start(0, 0)

        def chunk_body(ci, carry):
            b = lax.rem(ci, 2)
            wait(ci, b)

            @pl.when(ci + 1 < n_chunks)
            def _():
                start(ci + 1, 1 - b)

            base = ci * C

            @plsc.parallel_loop(0, C, unroll=8)
            def ebody(e):
                prods = []
                for j in range(JW):
                    sl = srow_v[b, e, pl.ds(j * L2, L2)]
                    dl = drow_v[b, e, pl.ds(j * L2, L2)]
                    prods.append(sl * dl)
                while len(prods) > 1:
                    prods = [x + y for x, y in zip(prods[::2], prods[1::2])]
                pa, pb = plsc.unpack(prods[0],
                                     format=plsc.PackFormat.INTERLEAVED)
                csum = plsc.cumsum(pa + pb)
                plsc.store_scatter(score_v,
                                   [jnp.full((L,), base + e, jnp.int32)],
                                   csum, mask=last_lane)

            return carry

        lax.fori_loop(0, n_chunks, chunk_body, 0)
        pltpu.sync_copy(score_v, out_hbm.at[wid])

    return sc_kernel


def kernel(h_src, h_dst, edge_index):
    N, D = h_src.shape
    E = edge_index.shape[1]
    Ep = -(-E // (NW * C)) * (NW * C)
    src_bf = h_src.astype(jnp.bfloat16)
    dst_bf = h_dst.astype(jnp.bfloat16)
    eidx = jnp.pad(edge_index, ((0, 0), (0, Ep - E)))
    eidx = eidx.reshape(2, NW, Ep // NW // G, G)
    out = _make_sc_kernel(Ep, N, D)(src_bf, dst_bf, eidx)
    return out.reshape(Ep)[:E]


# speedup vs baseline: 2.1395x; 2.1395x over previous
import functools

import jax
import jax.numpy as jnp
from jax import lax
from jax.experimental import pallas as pl
from jax.experimental.pallas import tpu as pltpu
from jax.experimental.pallas import tpu_sc as plsc

L = 16
L2 = 32
NW = 32
G = 128
C = 256


@functools.cache
def _make_sc_kernel(Ep, N, D):
    per_w = Ep // NW
    n_chunks = per_w // C
    JW = D // L2
    assert per_w % C == 0 and D % L2 == 0

    mesh = plsc.VectorSubcoreMesh(core_axis_name="c", subcore_axis_name="s")

    @functools.partial(
        pl.kernel,
        mesh=mesh,
        out_type=jax.ShapeDtypeStruct((NW, per_w), jnp.float32),
        compiler_params=pltpu.CompilerParams(needs_layout_passes=False,
                                             use_tc_tiling_on_sc=False),
        scratch_types=[
            pltpu.VMEM((2, per_w // G, G), jnp.int32),
            pltpu.VMEM((2, C, D), jnp.bfloat16),
            pltpu.VMEM((2, C, D), jnp.bfloat16),
            pltpu.VMEM((per_w,), jnp.float32),
            pltpu.SemaphoreType.DMA((2,)),
            pltpu.SemaphoreType.DMA((2,)),
        ],
    )
    def sc_kernel(hsrc_hbm, hdst_hbm, eidx_hbm, out_hbm,
                  eidx_v, srow_v, drow_v, score_v, ssem, dsem):
        wid = lax.axis_index("s") * 2 + lax.axis_index("c")
        pltpu.sync_copy(eidx_hbm.at[:, wid], eidx_v)

        last_lane = lax.iota(jnp.int32, L) == L - 1

        def start(ci, b):
            for h in range(2):
                pltpu.async_copy(hsrc_hbm.at[eidx_v.at[0, 2 * ci + h]],
                                 srow_v.at[b, pl.ds(h * G, G)], ssem.at[b])
                pltpu.async_copy(hdst_hbm.at[eidx_v.at[1, 2 * ci + h]],
                                 drow_v.at[b, pl.ds(h * G, G)], dsem.at[b])

        def wait(ci, b):
            for h in range(2):
                pltpu.make_async_copy(
                    hsrc_hbm.at[eidx_v.at[0, 2 * ci + h]],
                    srow_v.at[b, pl.ds(h * G, G)], ssem.at[b]).wait()
                pltpu.make_async_copy(
                    hdst_hbm.at[eidx_v.at[1, 2 * ci + h]],
                    drow_v.at[b, pl.ds(h * G, G)], dsem.at[b]).wait()

        start(0, 0)

        def chunk_body(ci, carry):
            b = lax.rem(ci, 2)
            wait(ci, b)

            @pl.when(ci + 1 < n_chunks)
            def _():
                start(ci + 1, 1 - b)

            base = ci * C

            @plsc.parallel_loop(0, C, unroll=8)
            def ebody(e):
                prods = []
                for j in range(JW):
                    sl = srow_v[b, e, pl.ds(j * L2, L2)]
                    dl = drow_v[b, e, pl.ds(j * L2, L2)]
                    prods.append(sl * dl)
                while len(prods) > 1:
                    prods = [x + y for x, y in zip(prods[::2], prods[1::2])]
                pa, pb = plsc.unpack(prods[0],
                                     format=plsc.PackFormat.INTERLEAVED)
                csum = plsc.cumsum(pa + pb)
                plsc.store_scatter(score_v,
                                   [jnp.full((L,), base + e, jnp.int32)],
                                   csum, mask=last_lane)

            return carry

        lax.fori_loop(0, n_chunks, chunk_body, 0)
        pltpu.sync_copy(score_v, out_hbm.at[wid])

    return sc_kernel


def kernel(h_src, h_dst, edge_index):
    N, D = h_src.shape
    E = edge_index.shape[1]
    Ep = -(-E // (NW * C)) * (NW * C)
    src_bf = h_src.astype(jnp.bfloat16)
    dst_bf = h_dst.astype(jnp.bfloat16)
    fill = jax.lax.broadcasted_iota(jnp.int32, (2, Ep - E), 1) % N
    eidx = jnp.concatenate([edge_index, fill], axis=1)
    eidx = eidx.reshape(2, NW, Ep // NW // G, G)
    out = _make_sc_kernel(Ep, N, D)(src_bf, dst_bf, eidx)
    return out.reshape(Ep)[:E]


# speedup vs baseline: 2.1798x; 1.0188x over previous
import functools

import jax
import jax.numpy as jnp
from jax import lax
from jax.experimental import pallas as pl
from jax.experimental.pallas import tpu as pltpu
from jax.experimental.pallas import tpu_sc as plsc

L = 16
L2 = 32
NW = 32
G = 128
C = 256


@functools.cache
def _make_sc_kernel(E, Ep, N, D):
    per_w = Ep // NW
    tail = E - (NW - 1) * per_w
    n_chunks = per_w // C
    JW = D // L2
    assert per_w % C == 0 and D % L2 == 0

    mesh = plsc.VectorSubcoreMesh(core_axis_name="c", subcore_axis_name="s")

    @functools.partial(
        pl.kernel,
        mesh=mesh,
        out_type=jax.ShapeDtypeStruct((E,), jnp.float32),
        compiler_params=pltpu.CompilerParams(needs_layout_passes=False,
                                             use_tc_tiling_on_sc=False),
        scratch_types=[
            pltpu.VMEM((2, per_w // G, G), jnp.int32),
            pltpu.VMEM((2, C, D), jnp.bfloat16),
            pltpu.VMEM((2, C, D), jnp.bfloat16),
            pltpu.VMEM((per_w,), jnp.float32),
            pltpu.SemaphoreType.DMA((2,)),
            pltpu.SemaphoreType.DMA((2,)),
        ],
    )
    def sc_kernel(hsrc_hbm, hdst_hbm, eidx_hbm, out_hbm,
                  eidx_v, srow_v, drow_v, score_v, ssem, dsem):
        wid = lax.axis_index("s") * 2 + lax.axis_index("c")
        pltpu.sync_copy(eidx_hbm.at[:, wid], eidx_v)

        last_lane = lax.iota(jnp.int32, L) == L - 1

        def start(ci, b):
            for h in range(2):
                pltpu.async_copy(hsrc_hbm.at[eidx_v.at[0, 2 * ci + h]],
                                 srow_v.at[b, pl.ds(h * G, G)], ssem.at[b])
                pltpu.async_copy(hdst_hbm.at[eidx_v.at[1, 2 * ci + h]],
                                 drow_v.at[b, pl.ds(h * G, G)], dsem.at[b])

        def wait(ci, b):
            for h in range(2):
                pltpu.make_async_copy(
                    hsrc_hbm.at[eidx_v.at[0, 2 * ci + h]],
                    srow_v.at[b, pl.ds(h * G, G)], ssem.at[b]).wait()
                pltpu.make_async_copy(
                    hdst_hbm.at[eidx_v.at[1, 2 * ci + h]],
                    drow_v.at[b, pl.ds(h * G, G)], dsem.at[b]).wait()

        start(0, 0)

        def chunk_body(ci, carry):
            b = lax.rem(ci, 2)
            wait(ci, b)

            @pl.when(ci + 1 < n_chunks)
            def _():
                start(ci + 1, 1 - b)

            base = ci * C

            @plsc.parallel_loop(0, C, unroll=8)
            def ebody(e):
                prods = []
                for j in range(JW):
                    sl = srow_v[b, e, pl.ds(j * L2, L2)]
                    dl = drow_v[b, e, pl.ds(j * L2, L2)]
                    prods.append(sl * dl)
                while len(prods) > 1:
                    prods = [x + y for x, y in zip(prods[::2], prods[1::2])]
                pa, pb = plsc.unpack(prods[0],
                                     format=plsc.PackFormat.INTERLEAVED)
                csum = plsc.cumsum(pa + pb)
                plsc.store_scatter(score_v,
                                   [jnp.full((L,), base + e, jnp.int32)],
                                   csum, mask=last_lane)

            return carry

        lax.fori_loop(0, n_chunks, chunk_body, 0)

        @pl.when(wid < NW - 1)
        def _():
            pltpu.sync_copy(score_v, out_hbm.at[pl.ds(wid * per_w, per_w)])

        @pl.when(wid == NW - 1)
        def _():
            pltpu.sync_copy(score_v.at[pl.ds(0, tail)],
                            out_hbm.at[pl.ds((NW - 1) * per_w, tail)])

    return sc_kernel


def kernel(h_src, h_dst, edge_index):
    N, D = h_src.shape
    E = edge_index.shape[1]
    Ep = -(-E // (NW * C)) * (NW * C)
    src_bf = h_src.astype(jnp.bfloat16)
    dst_bf = h_dst.astype(jnp.bfloat16)
    fill = jax.lax.broadcasted_iota(jnp.int32, (2, Ep - E), 1) % N
    eidx = jnp.concatenate([edge_index, fill], axis=1)
    eidx = eidx.reshape(2, NW, Ep // NW // G, G)
    return _make_sc_kernel(E, Ep, N, D)(src_bf, dst_bf, eidx)
